# Initial kernel scaffold; baseline (speedup 1.0000x reference)
import jax
import jax.numpy as jnp
from jax.experimental import pallas as pl


def kernel(x, edge_index, W1, b1, W2, b2):
    raise NotImplementedError("write your pallas kernel here")


# speedup vs baseline: 19.6708x; 19.6708x over previous
import functools
import math

import jax
import jax.numpy as jnp
from jax import lax
from jax.experimental import pallas as pl
from jax.experimental.pallas import tpu as pltpu
from jax.experimental.pallas import tpu_sc as plsc

_NC = 2
_NS = 16
_NW = _NC * _NS
_CHUNK = 128
_RBLK = 1280


def _mesh():
    return plsc.VectorSubcoreMesh(
        core_axis_name="c", subcore_axis_name="s",
        num_cores=_NC, num_subcores=_NS)


def _make_deg(n_pad, epw, nch):
    rpt = n_pad // _NS

    def body(dst, zinit, ones, out, acc, didx, ones_v):
        c = lax.axis_index("c")
        s = lax.axis_index("s")
        wid = s * _NC + c
        pltpu.sync_copy(zinit.at[pl.ds(s * rpt, rpt)],
                        acc.at[pl.ds(s * rpt, rpt)])
        pltpu.sync_copy(ones, ones_v)
        plsc.subcore_barrier()
        base = wid * epw

        def step(j, carry):
            pltpu.sync_copy(dst.at[pl.ds(base + j * _CHUNK, _CHUNK)],
                            didx.at[0])
            pltpu.sync_copy(ones_v, acc.at[didx.at[0]], add=True)
            return carry

        lax.fori_loop(0, nch, step, 0)
        plsc.subcore_barrier()
        pltpu.sync_copy(acc.at[pl.ds(s * rpt, rpt)],
                        out.at[c, pl.ds(s * rpt, rpt)])

    return pl.kernel(
        body,
        out_type=jax.ShapeDtypeStruct((_NC, n_pad, 16), jnp.float32),
        mesh=_mesh(),
        compiler_params=pltpu.CompilerParams(use_tc_tiling_on_sc=False),
        scratch_types=[
            pltpu.VMEM_SHARED((n_pad, 16), jnp.float32),
            pltpu.VMEM((1, _CHUNK), jnp.int32),
            pltpu.VMEM((_CHUNK, 16), jnp.float32),
        ],
    )


def _make_agg(n_pad, d, epw, nch):
    rpt = n_pad // _NS

    def body(table, src, dst, zinit, out, acc, sidx, didx, rows, sem):
        c = lax.axis_index("c")
        s = lax.axis_index("s")
        wid = s * _NC + c
        pltpu.sync_copy(zinit.at[pl.ds(s * rpt, rpt)],
                        acc.at[pl.ds(s * rpt, rpt)])
        plsc.subcore_barrier()
        base = wid * epw

        def step(j, carry):
            off = base + j * _CHUNK
            pltpu.sync_copy(src.at[pl.ds(off, _CHUNK)], sidx.at[0])
            pltpu.sync_copy(dst.at[pl.ds(off, _CHUNK)], didx.at[0])
            pltpu.async_copy(table.at[sidx.at[0]], rows, sem).wait()
            pltpu.sync_copy(rows, acc.at[didx.at[0]], add=True)
            return carry

        lax.fori_loop(0, nch, step, 0)
        plsc.subcore_barrier()
        pltpu.sync_copy(acc.at[pl.ds(s * rpt, rpt)],
                        out.at[c, pl.ds(s * rpt, rpt)])

    return pl.kernel(
        body,
        out_type=jax.ShapeDtypeStruct((_NC, n_pad, d), jnp.float32),
        mesh=_mesh(),
        compiler_params=pltpu.CompilerParams(use_tc_tiling_on_sc=False),
        scratch_types=[
            pltpu.VMEM_SHARED((n_pad, d), jnp.float32),
            pltpu.VMEM((1, _CHUNK), jnp.int32),
            pltpu.VMEM((1, _CHUNK), jnp.int32),
            pltpu.VMEM((_CHUNK, d), jnp.float32),
            pltpu.SemaphoreType.DMA,
        ],
    )


def _dinv(degp_ref):
    deg = 1.0 + degp_ref[0, :, 0:1] + degp_ref[1, :, 0:1]
    return lax.rsqrt(deg)


def _mm1_body(degp, x, w1, out):
    h = jnp.dot(x[:], w1[:], preferred_element_type=jnp.float32)
    out[:] = h * _dinv(degp)


def _mid_body(degp, agg, h1s, w2, b1, out):
    dinv = _dinv(degp)
    tot = agg[0] + agg[1] + h1s[:]
    a = jnp.maximum(tot * dinv + b1[:], 0.0)
    h2 = jnp.dot(a, w2[:], preferred_element_type=jnp.float32)
    out[:] = h2 * dinv


def _fin_body(degp, agg, h2s, b2, out):
    tot = agg[0] + agg[1] + h2s[:]
    out[:] = tot * _dinv(degp) + b2[:]


def _tc_call(body, grid, in_specs, out_spec, out_shape):
    return pl.pallas_call(
        body, grid=grid, in_specs=in_specs, out_specs=out_spec,
        out_shape=out_shape)


def kernel(x, edge_index, W1, b1, W2, b2):
    n, d_feat = x.shape
    e = edge_index.shape[1]
    d_hid = W1.shape[1]
    d_out = W2.shape[1]
    d2 = 16

    n_pad = math.ceil((n + 64) / _RBLK) * _RBLK
    epw = math.ceil(e / (_NW * _CHUNK)) * _CHUNK
    e_pad = epw * _NW
    nch = epw // _CHUNK
    grid = n_pad // _RBLK

    src = edge_index[0].astype(jnp.int32)
    dst = edge_index[1].astype(jnp.int32)
    n_garbage = n_pad - n
    pad_idx = n + jnp.arange(e_pad - e, dtype=jnp.int32) % n_garbage
    src = jnp.concatenate([src, pad_idx])
    dst = jnp.concatenate([dst, pad_idx])

    x_pad = jnp.pad(x, ((0, n_pad - n), (0, 0)))
    w2p = jnp.pad(W2, ((0, 0), (0, d2 - d_out)))
    b1r = b1.reshape(1, d_hid)
    b2p = jnp.pad(b2, (0, d2 - d_out)).reshape(1, d2)
    z16 = jnp.zeros((n_pad, 16), jnp.float32)
    z64 = jnp.zeros((n_pad, d_hid), jnp.float32)
    ones16 = jnp.ones((_CHUNK, 16), jnp.float32)

    degp = _make_deg(n_pad, epw, nch)(dst, z16, ones16)

    spec_degp = pl.BlockSpec((2, _RBLK, 16), lambda i: (0, i, 0))
    spec_row = lambda d: pl.BlockSpec((_RBLK, d), lambda i: (i, 0))
    spec_full = lambda shp: pl.BlockSpec(shp, lambda i: tuple(0 for _ in shp))

    h1s = _tc_call(
        _mm1_body, (grid,),
        [spec_degp, spec_row(d_feat), spec_full((d_feat, d_hid))],
        spec_row(d_hid), jax.ShapeDtypeStruct((n_pad, d_hid), jnp.float32),
    )(degp, x_pad, W1)

    agg1 = _make_agg(n_pad, d_hid, epw, nch)(h1s, src, dst, z64)

    h2s = _tc_call(
        _mid_body, (grid,),
        [spec_degp, pl.BlockSpec((2, _RBLK, d_hid), lambda i: (0, i, 0)),
         spec_row(d_hid), spec_full((d_hid, d2)), spec_full((1, d_hid))],
        spec_row(d2), jax.ShapeDtypeStruct((n_pad, d2), jnp.float32),
    )(degp, agg1, h1s, w2p, b1r)

    agg2 = _make_agg(n_pad, d2, epw, nch)(h2s, src, dst, z16)

    outp = _tc_call(
        _fin_body, (grid,),
        [spec_degp, spec_degp, spec_row(d2), spec_full((1, d2))],
        spec_row(d2), jax.ShapeDtypeStruct((n_pad, d2), jnp.float32),
    )(degp, agg2, h2s, b2p)

    return outp[:n, :d_out]


# speedup vs baseline: 43.7699x; 2.2251x over previous
import functools
import math

import jax
import jax.numpy as jnp
from jax import lax
from jax.experimental import pallas as pl
from jax.experimental.pallas import tpu as pltpu
from jax.experimental.pallas import tpu_sc as plsc

_NC = 2
_NS = 16
_NW = _NC * _NS
_CHUNK = 128
_RBLK = 1280


def _mesh():
    return plsc.VectorSubcoreMesh(
        core_axis_name="c", subcore_axis_name="s",
        num_cores=_NC, num_subcores=_NS)


def _make_deg(n_pad, nch):
    rpt = n_pad // _NS

    def body(dst, zinit, ones, out, acc, didx, ones_v):
        c = lax.axis_index("c")
        s = lax.axis_index("s")
        wid = s * _NC + c
        pltpu.sync_copy(zinit.at[pl.ds(s * rpt, rpt)],
                        acc.at[pl.ds(s * rpt, rpt)])
        pltpu.sync_copy(ones, ones_v)
        pltpu.sync_copy(dst.at[wid], didx)
        plsc.subcore_barrier()

        def step(j, carry):
            pltpu.sync_copy(ones_v, acc.at[didx.at[j]], add=True)
            return carry

        lax.fori_loop(0, nch, step, 0)
        plsc.subcore_barrier()
        pltpu.sync_copy(acc.at[pl.ds(s * rpt, rpt)],
                        out.at[c, pl.ds(s * rpt, rpt)])

    return pl.kernel(
        body,
        out_type=jax.ShapeDtypeStruct((_NC, n_pad, 16), jnp.float32),
        mesh=_mesh(),
        compiler_params=pltpu.CompilerParams(use_tc_tiling_on_sc=False),
        scratch_types=[
            pltpu.VMEM_SHARED((n_pad, 16), jnp.float32),
            pltpu.VMEM((nch, _CHUNK), jnp.int32),
            pltpu.VMEM((_CHUNK, 16), jnp.float32),
        ],
    )


def _make_agg(n_pad, d, nch, kbuf):
    rpt = n_pad // _NS
    ngrp = nch // kbuf

    def body(table, src, dst, zinit, out, acc, sidx, didx, rows, sem):
        c = lax.axis_index("c")
        s = lax.axis_index("s")
        wid = s * _NC + c
        pltpu.sync_copy(zinit.at[pl.ds(s * rpt, rpt)],
                        acc.at[pl.ds(s * rpt, rpt)])
        pltpu.sync_copy(src.at[wid], sidx)
        pltpu.sync_copy(dst.at[wid], didx)
        plsc.subcore_barrier()

        def group(g, carry):
            base = g * kbuf
            descs = [
                pltpu.async_copy(table.at[sidx.at[base + b]], rows.at[b], sem)
                for b in range(kbuf)
            ]
            for b in range(kbuf):
                descs[b].wait()
                pltpu.sync_copy(rows.at[b], acc.at[didx.at[base + b]],
                                add=True)
            return carry

        lax.fori_loop(0, ngrp, group, 0)
        plsc.subcore_barrier()
        pltpu.sync_copy(acc.at[pl.ds(s * rpt, rpt)],
                        out.at[c, pl.ds(s * rpt, rpt)])

    return pl.kernel(
        body,
        out_type=jax.ShapeDtypeStruct((_NC, n_pad, d), jnp.float32),
        mesh=_mesh(),
        compiler_params=pltpu.CompilerParams(use_tc_tiling_on_sc=False),
        scratch_types=[
            pltpu.VMEM_SHARED((n_pad, d), jnp.float32),
            pltpu.VMEM((nch, _CHUNK), jnp.int32),
            pltpu.VMEM((nch, _CHUNK), jnp.int32),
            pltpu.VMEM((kbuf, _CHUNK, d), jnp.float32),
            pltpu.SemaphoreType.DMA,
        ],
    )


def _dinv(degp_ref):
    deg = 1.0 + degp_ref[0, :, 0:1] + degp_ref[1, :, 0:1]
    return lax.rsqrt(deg)


def _mm1_body(degp, x, w1, out):
    h = jnp.dot(x[:], w1[:], preferred_element_type=jnp.float32)
    out[:] = h * _dinv(degp)


def _mid_body(degp, agg, h1s, w2, b1, out):
    dinv = _dinv(degp)
    tot = agg[0] + agg[1] + h1s[:]
    a = jnp.maximum(tot * dinv + b1[:], 0.0)
    h2 = jnp.dot(a, w2[:], preferred_element_type=jnp.float32)
    out[:] = h2 * dinv


def _fin_body(degp, agg, h2s, b2, out):
    tot = agg[0] + agg[1] + h2s[:]
    out[:] = tot * _dinv(degp) + b2[:]


def _tc_call(body, grid, in_specs, out_spec, out_shape):
    return pl.pallas_call(
        body, grid=grid, in_specs=in_specs, out_specs=out_spec,
        out_shape=out_shape)


def kernel(x, edge_index, W1, b1, W2, b2):
    n, d_feat = x.shape
    e = edge_index.shape[1]
    d_hid = W1.shape[1]
    d_out = W2.shape[1]
    d2 = 16

    kbuf = 8
    n_pad = math.ceil((n + 64) / _RBLK) * _RBLK
    epw = math.ceil(e / (_NW * kbuf * _CHUNK)) * kbuf * _CHUNK
    e_pad = epw * _NW
    nch = epw // _CHUNK
    grid = n_pad // _RBLK

    src = edge_index[0].astype(jnp.int32)
    dst = edge_index[1].astype(jnp.int32)
    n_garbage = n_pad - n
    pad_idx = n + jnp.arange(e_pad - e, dtype=jnp.int32) % n_garbage
    src = jnp.concatenate([src, pad_idx]).reshape(_NW, nch, _CHUNK)
    dst = jnp.concatenate([dst, pad_idx]).reshape(_NW, nch, _CHUNK)

    x_pad = jnp.pad(x, ((0, n_pad - n), (0, 0)))
    w2p = jnp.pad(W2, ((0, 0), (0, d2 - d_out)))
    b1r = b1.reshape(1, d_hid)
    b2p = jnp.pad(b2, (0, d2 - d_out)).reshape(1, d2)
    z16 = jnp.zeros((n_pad, 16), jnp.float32)
    z64 = jnp.zeros((n_pad, d_hid), jnp.float32)
    ones16 = jnp.ones((_CHUNK, 16), jnp.float32)

    degp = _make_deg(n_pad, nch)(dst, z16, ones16)

    spec_degp = pl.BlockSpec((2, _RBLK, 16), lambda i: (0, i, 0))
    spec_row = lambda d: pl.BlockSpec((_RBLK, d), lambda i: (i, 0))
    spec_full = lambda shp: pl.BlockSpec(shp, lambda i: tuple(0 for _ in shp))

    h1s = _tc_call(
        _mm1_body, (grid,),
        [spec_degp, spec_row(d_feat), spec_full((d_feat, d_hid))],
        spec_row(d_hid), jax.ShapeDtypeStruct((n_pad, d_hid), jnp.float32),
    )(degp, x_pad, W1)

    agg1 = _make_agg(n_pad, d_hid, nch, kbuf)(h1s, src, dst, z64)

    h2s = _tc_call(
        _mid_body, (grid,),
        [spec_degp, pl.BlockSpec((2, _RBLK, d_hid), lambda i: (0, i, 0)),
         spec_row(d_hid), spec_full((d_hid, d2)), spec_full((1, d_hid))],
        spec_row(d2), jax.ShapeDtypeStruct((n_pad, d2), jnp.float32),
    )(degp, agg1, h1s, w2p, b1r)

    agg2 = _make_agg(n_pad, d2, nch, kbuf)(h2s, src, dst, z16)

    outp = _tc_call(
        _fin_body, (grid,),
        [spec_degp, spec_degp, spec_row(d2), spec_full((1, d2))],
        spec_row(d2), jax.ShapeDtypeStruct((n_pad, d2), jnp.float32),
    )(degp, agg2, h2s, b2p)

    return outp[:n, :d_out]


# speedup vs baseline: 55.7031x; 1.2726x over previous
import functools
import math

import jax
import jax.numpy as jnp
from jax import lax
from jax.experimental import pallas as pl
from jax.experimental.pallas import tpu as pltpu
from jax.experimental.pallas import tpu_sc as plsc

_NC = 2
_NS = 16
_NW = _NC * _NS
_CHUNK = 128
_KBUF = 8


def _mesh():
    return plsc.VectorSubcoreMesh(
        core_axis_name="c", subcore_axis_name="s",
        num_cores=_NC, num_subcores=_NS)


def _zero_fill(buf, d):
    zv = jnp.zeros((16,), jnp.float32)

    def zrow(i, carry):
        for q in range(d // 16):
            buf[i, pl.ds(q * 16, 16)] = zv
        return carry

    lax.fori_loop(0, _CHUNK, zrow, 0)


def _zero_stripe(acc, src_buf, s, rpt):
    def zcp(i, carry):
        pltpu.sync_copy(src_buf,
                        acc.at[pl.ds(s * rpt + i * _CHUNK, _CHUNK)])
        return carry

    lax.fori_loop(0, rpt // _CHUNK, zcp, 0)


def _stage_idx(sd, sdv, wid, base_ch, extra):
    start = wid * base_ch + jnp.minimum(wid, extra)
    cnt = base_ch + jnp.where(wid < extra, 1, 0)
    pltpu.sync_copy(sd.at[pl.ds(start, base_ch)],
                    sdv.at[pl.ds(0, base_ch)])
    if extra:
        @pl.when(wid < extra)
        def _():
            pltpu.sync_copy(sd.at[pl.ds(start + base_ch, 1)],
                            sdv.at[pl.ds(base_ch, 1)])
    return cnt


def _make_deg(n_pad, base_ch, extra):
    rpt = n_pad // _NS
    cap = base_ch + (1 if extra else 0)
    ngrp = base_ch // _KBUF
    full = ngrp * _KBUF

    def body(sd, out16, out64, acc, sdv, ones_v, tmp16, rep, ssem):
        c = lax.axis_index("c")
        s = lax.axis_index("s")
        wid = s * _NC + c
        _zero_fill(ones_v, 16)
        _zero_stripe(acc, ones_v, s, rpt)
        ov = jnp.full((16,), 1.0, jnp.float32)

        def orow(i, carry):
            ones_v[i, pl.ds(0, 16)] = ov
            return carry

        lax.fori_loop(0, _CHUNK, orow, 0)
        cnt = _stage_idx(sd, sdv, wid, base_ch, extra)
        plsc.subcore_barrier()

        def group(g, carry):
            base = g * _KBUF
            descs = [
                pltpu.async_copy(ones_v, acc.at[sdv.at[base + b, 1]],
                                 ssem, add=True)
                for b in range(_KBUF)
            ]
            for dsc in descs:
                dsc.wait()
            return carry

        lax.fori_loop(0, ngrp, group, 0)

        def tstep(i, carry):
            pltpu.sync_copy(ones_v, acc.at[sdv.at[full + i, 1]], add=True)
            return carry

        lax.fori_loop(0, cnt - full, tstep, 0)
        plsc.subcore_barrier()
        pltpu.sync_copy(acc.at[pl.ds(s * rpt, rpt)], tmp16)

        def rrow(i, carry):
            v = tmp16[i, pl.ds(0, 16)]
            for q in range(4):
                rep[i, pl.ds(q * 16, 16)] = v
            return carry

        lax.fori_loop(0, rpt, rrow, 0)
        pltpu.sync_copy(tmp16, out16.at[c, pl.ds(s * rpt, rpt)])
        pltpu.sync_copy(rep, out64.at[c, pl.ds(s * rpt, rpt)])

    return pl.kernel(
        body,
        out_type=[
            jax.ShapeDtypeStruct((_NC, n_pad, 16), jnp.float32),
            jax.ShapeDtypeStruct((_NC, n_pad, 64), jnp.float32),
        ],
        mesh=_mesh(),
        compiler_params=pltpu.CompilerParams(use_tc_tiling_on_sc=False),
        scratch_types=[
            pltpu.VMEM_SHARED((n_pad, 16), jnp.float32),
            pltpu.VMEM((cap, 2, _CHUNK), jnp.int32),
            pltpu.VMEM((_CHUNK, 16), jnp.float32),
            pltpu.VMEM((n_pad // _NS, 16), jnp.float32),
            pltpu.VMEM((n_pad // _NS, 64), jnp.float32),
            pltpu.SemaphoreType.DMA,
        ],
    )


def _make_agg(n_pad, d, base_ch, extra):
    rpt = n_pad // _NS
    cap = base_ch + (1 if extra else 0)
    ngrp = base_ch // _KBUF
    full = ngrp * _KBUF

    def body(table, sd, out, acc, sdv, rows, gsem, ssem):
        c = lax.axis_index("c")
        s = lax.axis_index("s")
        wid = s * _NC + c
        _zero_fill(rows.at[0], d)
        _zero_stripe(acc, rows.at[0], s, rpt)
        cnt = _stage_idx(sd, sdv, wid, base_ch, extra)
        plsc.subcore_barrier()

        def group(g, carry):
            base = g * _KBUF
            gds = [
                pltpu.async_copy(table.at[sdv.at[base + b, 0]],
                                 rows.at[b], gsem)
                for b in range(_KBUF)
            ]
            sds = []
            for b in range(_KBUF):
                gds[b].wait()
                sds.append(
                    pltpu.async_copy(rows.at[b],
                                     acc.at[sdv.at[base + b, 1]],
                                     ssem, add=True))
            for dsc in sds:
                dsc.wait()
            return carry

        lax.fori_loop(0, ngrp, group, 0)

        def tstep(i, carry):
            j = full + i
            pltpu.async_copy(table.at[sdv.at[j, 0]], rows.at[0],
                             gsem).wait()
            pltpu.sync_copy(rows.at[0], acc.at[sdv.at[j, 1]], add=True)
            return carry

        lax.fori_loop(0, cnt - full, tstep, 0)
        plsc.subcore_barrier()
        pltpu.sync_copy(acc.at[pl.ds(s * rpt, rpt)],
                        out.at[c, pl.ds(s * rpt, rpt)])

    return pl.kernel(
        body,
        out_type=jax.ShapeDtypeStruct((_NC, n_pad, d), jnp.float32),
        mesh=_mesh(),
        compiler_params=pltpu.CompilerParams(use_tc_tiling_on_sc=False),
        scratch_types=[
            pltpu.VMEM_SHARED((n_pad, d), jnp.float32),
            pltpu.VMEM((cap, 2, _CHUNK), jnp.int32),
            pltpu.VMEM((_KBUF, _CHUNK, d), jnp.float32),
            pltpu.SemaphoreType.DMA,
            pltpu.SemaphoreType.DMA,
        ],
    )


def _mm1_body(deg64, x2, wbd, out):
    dp = lax.rsqrt(1.0 + deg64[0] + deg64[1])
    h = jnp.dot(x2[:], wbd[:], preferred_element_type=jnp.float32)
    out[:] = h * dp


def _mid_body(deg64, agg, h1s, w2bd8, b1p8, out):
    dp = lax.rsqrt(1.0 + deg64[0] + deg64[1])
    tot = agg[0] + agg[1] + h1s[:]
    a = jnp.maximum(tot * dp + b1p8[:], 0.0)
    out[:] = jnp.dot(a * dp, w2bd8[:], preferred_element_type=jnp.float32)


def _fin_body(deg16, agg, h2s, b2p, out):
    dinvp = lax.rsqrt(1.0 + deg16[0] + deg16[1])
    tot = agg[0] + agg[1] + h2s[:]
    out[:] = tot * dinvp + b2p[:]


def _tc_call(body, grid, in_specs, out_spec, out_shape):
    return pl.pallas_call(
        body, grid=grid, in_specs=in_specs, out_specs=out_spec,
        out_shape=out_shape)


def kernel(x, edge_index, W1, b1, W2, b2):
    n, d_feat = x.shape
    e = edge_index.shape[1]
    d_hid = W1.shape[1]
    d_out = W2.shape[1]
    d2 = 16

    n_pad = 10240
    nfull = e // _CHUNK
    base_ch = nfull // _NW
    extra = nfull % _NW

    ei = edge_index.astype(jnp.int32)
    sd = ei.reshape(2, nfull, _CHUNK).transpose(1, 0, 2)

    w2p = jnp.pad(W2, ((0, 0), (0, d2 - d_out)))
    wbd = jnp.kron(jnp.eye(2, dtype=jnp.float32), W1)
    w2bd8 = jnp.kron(jnp.eye(8, dtype=jnp.float32), w2p)
    b1p8 = jnp.tile(b1, 8).reshape(1, 8 * d_hid)
    b2p = jnp.tile(jnp.pad(b2, (0, d2 - d_out)), 8).reshape(1, 128)

    deg16, deg64 = _make_deg(n_pad, base_ch, extra)(sd)
    deg64v2 = deg64.reshape(2, n_pad // 2, 128)
    deg64v8 = deg64.reshape(2, n_pad // 8, 512)
    deg16v8 = deg16.reshape(2, n_pad // 8, 128)

    x2 = x.reshape(n // 2, 2 * d_feat)

    spec_full = lambda shp: pl.BlockSpec(shp, lambda i: tuple(0 for _ in shp))

    r2 = 640
    h1s_p = _tc_call(
        _mm1_body, ((n_pad // 2) // r2,),
        [pl.BlockSpec((2, r2, 128), lambda i: (0, i, 0)),
         pl.BlockSpec((r2, 2 * d_feat), lambda i: (i, 0)),
         spec_full((2 * d_feat, 2 * d_hid))],
        pl.BlockSpec((r2, 2 * d_hid), lambda i: (i, 0)),
        jax.ShapeDtypeStruct((n_pad // 2, 2 * d_hid), jnp.float32),
    )(deg64v2, x2, wbd)

    agg1 = _make_agg(n_pad, d_hid, base_ch, extra)(
        h1s_p.reshape(n_pad, d_hid), sd)

    r8 = 160
    h2s = _tc_call(
        _mid_body, ((n_pad // 8) // r8,),
        [pl.BlockSpec((2, r8, 512), lambda i: (0, i, 0)),
         pl.BlockSpec((2, r8, 512), lambda i: (0, i, 0)),
         pl.BlockSpec((r8, 512), lambda i: (i, 0)),
         spec_full((8 * d_hid, 8 * d2)), spec_full((1, 8 * d_hid))],
        pl.BlockSpec((r8, 8 * d2), lambda i: (i, 0)),
        jax.ShapeDtypeStruct((n_pad // 8, 8 * d2), jnp.float32),
    )(deg64v8, agg1.reshape(2, n_pad // 8, 8 * d_hid),
      h1s_p.reshape(n_pad // 8, 8 * d_hid), w2bd8, b1p8)

    agg2 = _make_agg(n_pad, d2, base_ch, extra)(
        h2s.reshape(n_pad, d2), sd)

    rf = 128
    outp = _tc_call(
        _fin_body, ((n_pad // 8) // rf,),
        [pl.BlockSpec((2, rf, 128), lambda i: (0, i, 0)),
         pl.BlockSpec((2, rf, 128), lambda i: (0, i, 0)),
         pl.BlockSpec((rf, 128), lambda i: (i, 0)),
         spec_full((1, 128))],
        pl.BlockSpec((rf, 128), lambda i: (i, 0)),
        jax.ShapeDtypeStruct((n_pad // 8, 128), jnp.float32),
    )(deg16v8, agg2.reshape(2, n_pad // 8, 128), h2s, b2p)

    return outp.reshape(n_pad, d2)[:n, :d_out]


# speedup vs baseline: 58.6220x; 1.0524x over previous
import functools
import math

import jax
import jax.numpy as jnp
from jax import lax
from jax.experimental import pallas as pl
from jax.experimental.pallas import tpu as pltpu
from jax.experimental.pallas import tpu_sc as plsc

_NC = 2
_NS = 16
_NW = _NC * _NS
_CHUNK = 128
_KBUF = 8


def _mesh():
    return plsc.VectorSubcoreMesh(
        core_axis_name="c", subcore_axis_name="s",
        num_cores=_NC, num_subcores=_NS)


def _zero_fill(buf, d):
    zv = jnp.zeros((16,), jnp.float32)

    def zrow(i, carry):
        for q in range(d // 16):
            buf[i, pl.ds(q * 16, 16)] = zv
        return carry

    lax.fori_loop(0, _CHUNK, zrow, 0)


def _zero_stripe(acc, src_buf, s, rpt):
    def zcp(i, carry):
        pltpu.sync_copy(src_buf,
                        acc.at[pl.ds(s * rpt + i * _CHUNK, _CHUNK)])
        return carry

    lax.fori_loop(0, rpt // _CHUNK, zcp, 0)


def _stage_idx(sd, sdv, wid, base_ch, extra):
    start = wid * base_ch + jnp.minimum(wid, extra)
    cnt = base_ch + jnp.where(wid < extra, 1, 0)
    pltpu.sync_copy(sd.at[pl.ds(start, base_ch)],
                    sdv.at[pl.ds(0, base_ch)])
    if extra:
        @pl.when(wid < extra)
        def _():
            pltpu.sync_copy(sd.at[pl.ds(start + base_ch, 1)],
                            sdv.at[pl.ds(base_ch, 1)])
    return cnt


def _make_deg(n_pad, base_ch, extra):
    rpt = n_pad // _NS
    cap = base_ch + (1 if extra else 0)
    ngrp = base_ch // _KBUF
    full = ngrp * _KBUF

    def body(sd, out16, out64, acc, sdv, ones_v, tmp16, rep, ssem):
        c = lax.axis_index("c")
        s = lax.axis_index("s")
        wid = s * _NC + c
        _zero_fill(ones_v, 16)
        _zero_stripe(acc, ones_v, s, rpt)
        ov = jnp.full((16,), 1.0, jnp.float32)

        def orow(i, carry):
            ones_v[i, pl.ds(0, 16)] = ov
            return carry

        lax.fori_loop(0, _CHUNK, orow, 0)
        cnt = _stage_idx(sd, sdv, wid, base_ch, extra)
        plsc.subcore_barrier()

        def group(g, carry):
            base = g * _KBUF

            @pl.when(g >= 2)
            def _():
                for b in range(_KBUF):
                    pltpu.make_async_copy(
                        ones_v, acc.at[sdv.at[base + b, 1]], ssem).wait()

            for b in range(_KBUF):
                pltpu.async_copy(ones_v, acc.at[sdv.at[base + b, 1]],
                                 ssem, add=True)
            return carry

        lax.fori_loop(0, ngrp, group, 0)
        if ngrp >= 2:
            for b in range(2 * _KBUF):
                pltpu.make_async_copy(ones_v, acc.at[sdv.at[b, 1]],
                                      ssem).wait()

        def tstep(i, carry):
            pltpu.sync_copy(ones_v, acc.at[sdv.at[full + i, 1]], add=True)
            return carry

        lax.fori_loop(0, cnt - full, tstep, 0)
        plsc.subcore_barrier()
        pltpu.sync_copy(acc.at[pl.ds(s * rpt, rpt)], tmp16)

        def rrow(i, carry):
            v = tmp16[i, pl.ds(0, 16)]
            for q in range(4):
                rep[i, pl.ds(q * 16, 16)] = v
            return carry

        lax.fori_loop(0, rpt, rrow, 0)
        pltpu.sync_copy(tmp16, out16.at[c, pl.ds(s * rpt, rpt)])
        pltpu.sync_copy(rep, out64.at[c, pl.ds(s * rpt, rpt)])

    return pl.kernel(
        body,
        out_type=[
            jax.ShapeDtypeStruct((_NC, n_pad, 16), jnp.float32),
            jax.ShapeDtypeStruct((_NC, n_pad, 64), jnp.float32),
        ],
        mesh=_mesh(),
        compiler_params=pltpu.CompilerParams(use_tc_tiling_on_sc=False),
        scratch_types=[
            pltpu.VMEM_SHARED((n_pad, 16), jnp.float32),
            pltpu.VMEM((cap, 2, _CHUNK), jnp.int32),
            pltpu.VMEM((_CHUNK, 16), jnp.float32),
            pltpu.VMEM((n_pad // _NS, 16), jnp.float32),
            pltpu.VMEM((n_pad // _NS, 64), jnp.float32),
            pltpu.SemaphoreType.DMA,
        ],
    )


def _make_agg(n_pad, d, base_ch, extra, kbuf):
    rpt = n_pad // _NS
    cap = base_ch + (1 if extra else 0)
    ngrp = base_ch // kbuf
    full = ngrp * kbuf
    assert full == base_ch, (base_ch, kbuf)

    def body(table, sd, out, acc, sdv, rows, gsem, ssem):
        c = lax.axis_index("c")
        s = lax.axis_index("s")
        wid = s * _NC + c
        _zero_fill(rows.at[0], d)
        _zero_stripe(acc, rows.at[0], s, rpt)
        cnt = _stage_idx(sd, sdv, wid, base_ch, extra)
        plsc.subcore_barrier()

        def group(g, carry):
            base = g * kbuf
            off = (g % 2) * kbuf

            @pl.when(g >= 2)
            def _():
                for b in range(kbuf):
                    pltpu.make_async_copy(
                        rows.at[off + b], acc.at[sdv.at[base + b, 1]],
                        ssem).wait()

            gds = [
                pltpu.async_copy(table.at[sdv.at[base + b, 0]],
                                 rows.at[off + b], gsem)
                for b in range(kbuf)
            ]
            for b in range(kbuf):
                gds[b].wait()
                pltpu.async_copy(rows.at[off + b],
                                 acc.at[sdv.at[base + b, 1]],
                                 ssem, add=True)
            return carry

        lax.fori_loop(0, ngrp, group, 0)
        for b in range(2 * kbuf):
            pltpu.make_async_copy(rows.at[b], acc.at[sdv.at[b, 1]],
                                  ssem).wait()

        def tstep(i, carry):
            j = full + i
            pltpu.async_copy(table.at[sdv.at[j, 0]], rows.at[0],
                             gsem).wait()
            pltpu.sync_copy(rows.at[0], acc.at[sdv.at[j, 1]], add=True)
            return carry

        lax.fori_loop(0, cnt - full, tstep, 0)
        plsc.subcore_barrier()
        pltpu.sync_copy(acc.at[pl.ds(s * rpt, rpt)],
                        out.at[c, pl.ds(s * rpt, rpt)])

    return pl.kernel(
        body,
        out_type=jax.ShapeDtypeStruct((_NC, n_pad, d), jnp.float32),
        mesh=_mesh(),
        compiler_params=pltpu.CompilerParams(use_tc_tiling_on_sc=False),
        scratch_types=[
            pltpu.VMEM_SHARED((n_pad, d), jnp.float32),
            pltpu.VMEM((cap, 2, _CHUNK), jnp.int32),
            pltpu.VMEM((2 * kbuf, _CHUNK, d), jnp.float32),
            pltpu.SemaphoreType.DMA,
            pltpu.SemaphoreType.DMA,
        ],
    )


def _mm1_body(deg64, x2, wbd, out):
    dp = lax.rsqrt(1.0 + deg64[0] + deg64[1])
    h = jnp.dot(x2[:], wbd[:], preferred_element_type=jnp.float32)
    out[:] = h * dp


def _mid_body(deg64, agg, h1s, w2bd8, b1p8, out):
    dp = lax.rsqrt(1.0 + deg64[0] + deg64[1])
    tot = agg[0] + agg[1] + h1s[:]
    a = jnp.maximum(tot * dp + b1p8[:], 0.0)
    out[:] = jnp.dot(a * dp, w2bd8[:], preferred_element_type=jnp.float32)


def _fin_body(deg16, agg, h2s, b2p, out):
    dinvp = lax.rsqrt(1.0 + deg16[0] + deg16[1])
    tot = agg[0] + agg[1] + h2s[:]
    out[:] = tot * dinvp + b2p[:]


def _tc_call(body, grid, in_specs, out_spec, out_shape):
    return pl.pallas_call(
        body, grid=grid, in_specs=in_specs, out_specs=out_spec,
        out_shape=out_shape)


def kernel(x, edge_index, W1, b1, W2, b2):
    n, d_feat = x.shape
    e = edge_index.shape[1]
    d_hid = W1.shape[1]
    d_out = W2.shape[1]
    d2 = 16

    n_pad = 10240
    nfull = e // _CHUNK
    base_ch = nfull // _NW
    extra = nfull % _NW

    ei = edge_index.astype(jnp.int32)
    sd = ei.reshape(2, nfull, _CHUNK).transpose(1, 0, 2)

    w2p = jnp.pad(W2, ((0, 0), (0, d2 - d_out)))
    wbd = jnp.kron(jnp.eye(2, dtype=jnp.float32), W1)
    w2bd8 = jnp.kron(jnp.eye(8, dtype=jnp.float32), w2p)
    b1p8 = jnp.tile(b1, 8).reshape(1, 8 * d_hid)
    b2p = jnp.tile(jnp.pad(b2, (0, d2 - d_out)), 8).reshape(1, 128)

    deg16, deg64 = _make_deg(n_pad, base_ch, extra)(sd)
    deg64v2 = deg64.reshape(2, n_pad // 2, 128)
    deg64v8 = deg64.reshape(2, n_pad // 8, 512)
    deg16v8 = deg16.reshape(2, n_pad // 8, 128)

    x2 = x.reshape(n // 2, 2 * d_feat)

    spec_full = lambda shp: pl.BlockSpec(shp, lambda i: tuple(0 for _ in shp))

    r2 = 640
    h1s_p = _tc_call(
        _mm1_body, ((n_pad // 2) // r2,),
        [pl.BlockSpec((2, r2, 128), lambda i: (0, i, 0)),
         pl.BlockSpec((r2, 2 * d_feat), lambda i: (i, 0)),
         spec_full((2 * d_feat, 2 * d_hid))],
        pl.BlockSpec((r2, 2 * d_hid), lambda i: (i, 0)),
        jax.ShapeDtypeStruct((n_pad // 2, 2 * d_hid), jnp.float32),
    )(deg64v2, x2, wbd)

    agg1 = _make_agg(n_pad, d_hid, base_ch, extra, 3)(
        h1s_p.reshape(n_pad, d_hid), sd)

    r8 = 160
    h2s = _tc_call(
        _mid_body, ((n_pad // 8) // r8,),
        [pl.BlockSpec((2, r8, 512), lambda i: (0, i, 0)),
         pl.BlockSpec((2, r8, 512), lambda i: (0, i, 0)),
         pl.BlockSpec((r8, 512), lambda i: (i, 0)),
         spec_full((8 * d_hid, 8 * d2)), spec_full((1, 8 * d_hid))],
        pl.BlockSpec((r8, 8 * d2), lambda i: (i, 0)),
        jax.ShapeDtypeStruct((n_pad // 8, 8 * d2), jnp.float32),
    )(deg64v8, agg1.reshape(2, n_pad // 8, 8 * d_hid),
      h1s_p.reshape(n_pad // 8, 8 * d_hid), w2bd8, b1p8)

    agg2 = _make_agg(n_pad, d2, base_ch, extra, 6)(
        h2s.reshape(n_pad, d2), sd)

    rf = 128
    outp = _tc_call(
        _fin_body, ((n_pad // 8) // rf,),
        [pl.BlockSpec((2, rf, 128), lambda i: (0, i, 0)),
         pl.BlockSpec((2, rf, 128), lambda i: (0, i, 0)),
         pl.BlockSpec((rf, 128), lambda i: (i, 0)),
         spec_full((1, 128))],
        pl.BlockSpec((rf, 128), lambda i: (i, 0)),
        jax.ShapeDtypeStruct((n_pad // 8, 128), jnp.float32),
    )(deg16v8, agg2.reshape(2, n_pad // 8, 128), h2s, b2p)

    return outp.reshape(n_pad, d2)[:n, :d_out]
